# baseline (speedup 1.0000x reference)
import functools

import jax
import jax.numpy as jnp
from jax import lax
from jax.experimental import pallas as pl
from jax.experimental.pallas import tpu as pltpu
from jax.experimental.pallas import tpu_sc as plsc

_NUM_CLASSES = 1000000
_HIDDEN = 64
_BATCH = 16384

_INFO = plsc.get_sparse_core_info()
_NC = _INFO.num_cores
_NS = _INFO.num_subcores
_L = _INFO.num_lanes
_NW = _NC * _NS
_B_PER_W = _BATCH // _NW
_CHUNK = 128
_NCHUNK = _B_PER_W // _CHUNK
_NGROUP = _B_PER_W // _L

_mesh = plsc.VectorSubcoreMesh(core_axis_name="c", subcore_axis_name="s")


@functools.partial(
    pl.kernel,
    mesh=_mesh,
    out_type=jax.ShapeDtypeStruct((_BATCH, _HIDDEN), jnp.float32),
    scratch_types=[
        pltpu.VMEM((_NCHUNK, _CHUNK), jnp.int32),
        pltpu.VMEM((_B_PER_W,), jnp.int32),
        pltpu.VMEM((_B_PER_W,), jnp.int32),
        pltpu.VMEM((_B_PER_W // 2, 2 * _HIDDEN), jnp.float32),
        pltpu.VMEM((_B_PER_W, _HIDDEN), jnp.float32),
        pltpu.VMEM((_HIDDEN,), jnp.float32),
        pltpu.SemaphoreType.DMA,
    ],
    compiler_params=pltpu.CompilerParams(needs_layout_passes=False),
)
def _embed(labels_hbm, drop_hbm, pairs_hbm, null_hbm, out_hbm,
           idx_v, hof_v, drop_v, rows_v, out_v, null_v, sem):
    wid = lax.axis_index("s") * _NC + lax.axis_index("c")
    base = wid * _B_PER_W

    pltpu.sync_copy(drop_hbm.at[pl.ds(base, _B_PER_W)], drop_v)
    for j in range(_NCHUNK):
        pltpu.sync_copy(
            labels_hbm.at[pl.ds(base + j * _CHUNK, _CHUNK)], idx_v.at[j]
        )

    cnt = jnp.zeros((_L,), jnp.int32)
    for j in range(_NCHUNK):
        row = idx_v.at[j]
        for i in range(_CHUNK // _L):
            sl = pl.ds(i * _L, _L)
            g = j * (_CHUNK // _L) + i
            dr = drop_v[pl.ds(g * _L, _L)]
            r = jnp.where(dr != 0, _NUM_CLASSES, row[sl])
            p = jnp.minimum(r >> 1, _NUM_CLASSES // 2 - 1)
            hof_v[pl.ds(g * _L, _L)] = (r & 1) * _HIDDEN
            row[sl] = p
            cnt = cnt + dr
    n_drop = jnp.sum(cnt, axis=0)

    iota = lax.iota(jnp.int32, _L)
    half_chunks = _NCHUNK // 2
    half_rows = _B_PER_W // 2
    for ph in range(2):
        copies = [
            pltpu.async_copy(
                pairs_hbm.at[idx_v.at[ph * half_chunks + j]],
                rows_v.at[pl.ds(j * _CHUNK, _CHUNK)],
                sem,
            )
            for j in range(half_chunks)
        ]
        for c in copies:
            c.wait()
        def compact_group(gl, _, ph=ph):
            g = ph * (half_rows // _L) + gl
            row_idx = iota + g * _L
            loc_idx = iota + gl * _L
            hof = hof_v[pl.ds(g * _L, _L)]
            for c in range(_HIDDEN):
                col = jnp.full((_L,), c, jnp.int32)
                x = plsc.load_gather(rows_v, [loc_idx, hof + c])
                plsc.store_scatter(out_v, [row_idx, col], x)
            return 0

        lax.fori_loop(0, half_rows // _L, compact_group, 0)

    @pl.when(n_drop > 0)
    def _drop_fixup():
        pltpu.sync_copy(null_hbm, null_v)

        def fix_group(g, _):
            row_idx = iota + g * _L
            dmask = drop_v[pl.ds(g * _L, _L)] != 0
            for c in range(_HIDDEN):
                col = jnp.full((_L,), c, jnp.int32)
                x = plsc.load_gather(null_v, [col])
                plsc.store_scatter(out_v, [row_idx, col], x, mask=dmask)
            return 0

        lax.fori_loop(0, _NGROUP, fix_group, 0)

    pltpu.sync_copy(out_v, out_hbm.at[pl.ds(base, _B_PER_W)])


def kernel(labels, force_drop_ids, embedding_table):
    lbl = labels.astype(jnp.int32)
    drop = force_drop_ids.astype(jnp.int32)
    pairs = embedding_table[:_NUM_CLASSES].reshape(_NUM_CLASSES // 2,
                                                   2 * _HIDDEN)
    null_row = embedding_table[_NUM_CLASSES]
    return _embed(lbl, drop, pairs, null_row)


# speedup vs baseline: 1.8062x; 1.8062x over previous
import functools

import jax
import jax.numpy as jnp
from jax import lax
from jax.experimental import pallas as pl
from jax.experimental.pallas import tpu as pltpu
from jax.experimental.pallas import tpu_sc as plsc

_NUM_CLASSES = 1000000
_HIDDEN = 64
_BATCH = 16384

_INFO = plsc.get_sparse_core_info()
_NC = _INFO.num_cores
_NS = _INFO.num_subcores
_L = _INFO.num_lanes
_NW = _NC * _NS
_B_PER_W = _BATCH // _NW

_mesh = plsc.VectorSubcoreMesh(core_axis_name="c", subcore_axis_name="s")


@functools.partial(
    pl.kernel,
    mesh=_mesh,
    out_type=jax.ShapeDtypeStruct((_BATCH, _HIDDEN), jnp.float32),
    scratch_types=[
        pltpu.SMEM((_B_PER_W,), jnp.int32),
        pltpu.SMEM((_B_PER_W,), jnp.int32),
        pltpu.VMEM_SHARED((_NW, _B_PER_W), jnp.int32),
        pltpu.VMEM_SHARED((_NW, _B_PER_W), jnp.int32),
        pltpu.VMEM((_B_PER_W, _HIDDEN), jnp.float32),
        pltpu.SemaphoreType.DMA,
    ],
)
def _embed(labels_hbm, drop_hbm, table_hbm, out_hbm,
           lbl_s, drop_s, lbl_sp, drop_sp, rows_v, sem):
    wid = lax.axis_index("s") * _NC + lax.axis_index("c")
    base = wid * _B_PER_W
    with jax.named_scope("stage_idx"):
        pltpu.sync_copy(labels_hbm.at[pl.ds(base, _B_PER_W)], lbl_sp.at[wid])
        pltpu.sync_copy(drop_hbm.at[pl.ds(base, _B_PER_W)], drop_sp.at[wid])
        pltpu.sync_copy(lbl_sp.at[wid], lbl_s)
        pltpu.sync_copy(drop_sp.at[wid], drop_s)

    def body(i, _):
        r = lax.select(drop_s[i] != 0, _NUM_CLASSES, lbl_s[i])
        pltpu.async_copy(
            table_hbm.at[pl.ds(r, 1)],
            rows_v.at[pl.ds(i, 1)],
            sem,
        )
        return 0

    with jax.named_scope("row_streams"):
        lax.fori_loop(0, _B_PER_W, body, 0)
    with jax.named_scope("drain"):
        pltpu.make_async_copy(table_hbm.at[pl.ds(0, _B_PER_W)], rows_v,
                              sem).wait()
    with jax.named_scope("out_write"):
        pltpu.sync_copy(rows_v, out_hbm.at[pl.ds(base, _B_PER_W)])


def kernel(labels, force_drop_ids, embedding_table):
    lbl = labels.astype(jnp.int32)
    drop = force_drop_ids.astype(jnp.int32)
    return _embed(lbl, drop, embedding_table)
